# baseline (speedup 1.0000x reference)
import functools

import jax
import jax.numpy as jnp
from jax import lax
from jax.experimental import pallas as pl
from jax.experimental.pallas import tpu as pltpu
from jax.experimental.pallas import tpu_sc as plsc

_B, _HW, _C = 32, 1024, 768
_B_TC = 24
_B_SC = _B - _B_TC


def _tc_broadcast(table, n_batches):
    def body(tbl_ref, out_ref, sem):
        copies = [
            pltpu.make_async_copy(tbl_ref, out_ref.at[i], sem)
            for i in range(n_batches)
        ]
        for c in copies:
            c.start()
        for c in copies:
            c.wait()

    return pl.pallas_call(
        body,
        in_specs=[pl.BlockSpec(memory_space=pltpu.VMEM)],
        out_specs=pl.BlockSpec(memory_space=pl.ANY),
        out_shape=jax.ShapeDtypeStruct((n_batches, _HW, _C), jnp.float32),
        scratch_shapes=[pltpu.SemaphoreType.DMA],
    )(table)


def _sc_broadcast(table, n_batches):
    mesh = plsc.VectorSubcoreMesh(core_axis_name="c", subcore_axis_name="s")
    info = plsc.get_sparse_core_info()
    num_cores = info.num_cores
    num_subcores = info.num_subcores
    n_workers = num_cores * num_subcores
    stage_rows = _HW // num_subcores
    subs_per_batch = n_workers // n_batches
    rows_per_sub = _HW // subs_per_batch

    @functools.partial(
        pl.kernel,
        mesh=mesh,
        out_type=jax.ShapeDtypeStruct((n_batches, _HW, _C), jnp.float32),
        scratch_types=[
            pltpu.VMEM_SHARED((_HW, _C), jnp.float32),
            pltpu.SemaphoreType.DMA,
        ],
    )
    def k(table_hbm, out_hbm, shared, sem):
        sid = lax.axis_index("s")
        wid = sid * num_cores + lax.axis_index("c")
        row0 = sid * stage_rows
        pltpu.sync_copy(
            table_hbm.at[pl.ds(row0, stage_rows)],
            shared.at[pl.ds(row0, stage_rows)],
        )
        plsc.subcore_barrier()
        b = wid // subs_per_batch
        r0 = (wid % subs_per_batch) * rows_per_sub
        pltpu.sync_copy(
            shared.at[pl.ds(r0, rows_per_sub)],
            out_hbm.at[b, pl.ds(r0, rows_per_sub)],
        )

    return k(table)


def kernel(inputs, table):
    del inputs
    tc_part = _tc_broadcast(table, _B_TC)
    sc_part = _sc_broadcast(table, _B_SC)
    return jnp.concatenate([tc_part, sc_part], axis=0)


# speedup vs baseline: 2.0441x; 2.0441x over previous
import functools

import jax
import jax.numpy as jnp
from jax import lax
from jax.experimental import pallas as pl
from jax.experimental.pallas import tpu as pltpu
from jax.experimental.pallas import tpu_sc as plsc

_B, _HW, _C = 32, 1024, 768


def _sc_gather(table):
    mesh = plsc.VectorSubcoreMesh(core_axis_name="c", subcore_axis_name="s")
    info = plsc.get_sparse_core_info()
    num_cores = info.num_cores
    num_subcores = info.num_subcores
    n_workers = num_cores * num_subcores
    rows_per_sub = _HW // n_workers

    @functools.partial(
        pl.kernel,
        mesh=mesh,
        out_type=jax.ShapeDtypeStruct((_HW, _C), jnp.float32),
        scratch_types=[pltpu.VMEM((rows_per_sub, _C), jnp.float32)],
    )
    def k(table_hbm, emb_hbm, buf):
        wid = lax.axis_index("s") * num_cores + lax.axis_index("c")
        row0 = wid * rows_per_sub
        pltpu.sync_copy(table_hbm.at[pl.ds(row0, rows_per_sub)], buf)
        pltpu.sync_copy(buf, emb_hbm.at[pl.ds(row0, rows_per_sub)])

    return k(table)


def _tc_broadcast(emb):
    def body(emb_ref, out_ref, sem):
        copies = [
            pltpu.make_async_copy(emb_ref, out_ref.at[i], sem)
            for i in range(_B)
        ]
        for c in copies:
            c.start()
        for c in copies:
            c.wait()

    return pl.pallas_call(
        body,
        in_specs=[pl.BlockSpec(memory_space=pltpu.VMEM)],
        out_specs=pl.BlockSpec(memory_space=pl.ANY),
        out_shape=jax.ShapeDtypeStruct((_B, _HW, _C), jnp.float32),
        scratch_shapes=[pltpu.SemaphoreType.DMA],
    )(emb)


def kernel(inputs, table):
    del inputs
    return _tc_broadcast(_sc_gather(table))
